# baseline (speedup 1.0000x reference)
import functools

import jax
import jax.numpy as jnp
from jax import lax
from jax.experimental import pallas as pl
from jax.experimental.pallas import tpu as pltpu
from jax.experimental.pallas import tpu_sc as plsc

NC = 2
NS = 16
BATCH = 128


def _feat_body(x_ref, w_ref, b_ref, o_ref):
    acc = lax.dot_general(x_ref[...], w_ref[...], (((1,), (1,)), ((), ())),
                          preferred_element_type=jnp.float32)
    o_ref[...] = jnp.maximum(acc + b_ref[...], 0.0)


def _out_body(a_ref, c_ref, f_ref, wl_ref, bl_ref, wr_ref, o_ref):
    agg = a_ref[0] + a_ref[1]
    mean = agg / jnp.maximum(c_ref[...], 1.0)
    o_ref[...] = (
        lax.dot_general(mean, wl_ref[...], (((1,), (1,)), ((), ())),
                        preferred_element_type=jnp.float32)
        + bl_ref[...]
        + lax.dot_general(f_ref[...], wr_ref[...], (((1,), (1,)), ((), ())),
                          preferred_element_type=jnp.float32)
    )


def _make_sc_agg(n_pad, k0, k1, d):
    rows_per_sub = n_pad // NS
    kmax = max(k0, k1)
    mesh = plsc.VectorSubcoreMesh(core_axis_name="c", subcore_axis_name="s")

    @functools.partial(
        pl.kernel,
        mesh=mesh,
        out_type=[
            jax.ShapeDtypeStruct((NC, n_pad, d), jnp.float32),
            jax.ShapeDtypeStruct((n_pad,), jnp.float32),
            jax.ShapeDtypeStruct((n_pad,), jnp.float32),
        ],
        scratch_types=[
            pltpu.VMEM((kmax, BATCH), jnp.int32),
            pltpu.VMEM((BATCH,), jnp.int32),
            pltpu.VMEM((BATCH,), jnp.int32),
            pltpu.VMEM((BATCH,), jnp.int32),
            pltpu.VMEM((BATCH,), jnp.int32),
            pltpu.VMEM((BATCH, d), jnp.float32),
            pltpu.VMEM((BATCH, d), jnp.float32),
            pltpu.VMEM((BATCH,), jnp.float32),
            pltpu.VMEM((rows_per_sub,), jnp.float32),
            pltpu.VMEM_SHARED((n_pad, d), jnp.float32),
            pltpu.VMEM_SHARED((n_pad,), jnp.float32),
            pltpu.SemaphoreType.DMA,
            pltpu.SemaphoreType.DMA,
        ],
    )
    def sc_agg(feat_h, comb0_h, comb1_h,
               agg_h, cnt0_h, cnt1_h,
               comb_v, srcA, dstA, srcB, dstB, rows_v, rows2_v, ones_v,
               cstage_v, acc_sh, cnt_sh, sem, sem2):
        c = lax.axis_index("c")
        s = lax.axis_index("s")
        row0 = s * rows_per_sub
        zv = jnp.zeros((16,), jnp.float32)
        ov = jnp.ones((16,), jnp.float32)

        def zfill(i, carry):
            for cc in range(d // 16):
                rows_v[i, pl.ds(cc * 16, 16)] = zv
            ones_v[pl.ds(i * 16, 16)] = ov
            return carry
        lax.fori_loop(0, BATCH // 16, zfill, 0)

        def zfill2(i, carry):
            for cc in range(d // 16):
                rows_v[i, pl.ds(cc * 16, 16)] = zv
            return carry
        lax.fori_loop(BATCH // 16, BATCH, zfill2, 0)

        full, rem = divmod(rows_per_sub, BATCH)
        for b in range(full):
            pltpu.sync_copy(rows_v, acc_sh.at[pl.ds(row0 + b * BATCH, BATCH)])
        if rem:
            pltpu.sync_copy(rows_v.at[pl.ds(0, rem)],
                            acc_sh.at[pl.ds(row0 + full * BATCH, rem)])
        for b in range(rows_per_sub // BATCH):
            pltpu.sync_copy(rows_v.at[0],
                            cnt_sh.at[pl.ds(row0 + b * BATCH, BATCH)])
        if rows_per_sub % BATCH:
            pltpu.sync_copy(rows_v.at[0, pl.ds(0, rows_per_sub % BATCH)],
                            cnt_sh.at[pl.ds(row0 + full * BATCH,
                                            rows_per_sub % BATCH)])

        plsc.subcore_barrier()

        def prep(j, srci, dsti):
            for b in range(BATCH // 16):
                cmb = comb_v[j, pl.ds(b * 16, 16)]
                srci[pl.ds(b * 16, 16)] = jnp.bitwise_and(cmb, 0xFFFF)
                dsti[pl.ds(b * 16, 16)] = lax.shift_right_logical(cmb, 16)

        def wait_gather(buf, s_):
            pltpu.make_async_copy(feat_h.at[pl.ds(0, BATCH)], buf, s_).wait()

        def scatter(buf, dsti):
            pltpu.sync_copy(buf, acc_sh.at[dsti], add=True)
            pltpu.sync_copy(ones_v, cnt_sh.at[dsti], add=True)

        pltpu.sync_copy(comb0_h.at[c, s], comb_v)
        prep(0, srcA, dstA)
        pltpu.async_copy(feat_h.at[srcA], rows_v, sem)

        def pair(jj, carry):
            j0 = 2 * jj
            prep(j0 + 1, srcB, dstB)
            pltpu.async_copy(feat_h.at[srcB], rows2_v, sem2)
            wait_gather(rows_v, sem)
            scatter(rows_v, dstA)
            prep(jnp.minimum(j0 + 2, kmax - 1), srcA, dstA)
            pltpu.async_copy(feat_h.at[srcA], rows_v, sem)
            wait_gather(rows2_v, sem2)
            scatter(rows2_v, dstB)
            return carry

        lax.fori_loop(0, kmax // 2, pair, 0)
        wait_gather(rows_v, sem)
        plsc.subcore_barrier()
        pltpu.sync_copy(acc_sh.at[pl.ds(row0, rows_per_sub)],
                        agg_h.at[c, pl.ds(row0, rows_per_sub)])

        pltpu.sync_copy(cnt_sh.at[pl.ds(row0, rows_per_sub)], cstage_v)

        @pl.when(c == 0)
        def _():
            pltpu.sync_copy(cstage_v, cnt0_h.at[pl.ds(row0, rows_per_sub)])

        @pl.when(c == 1)
        def _():
            pltpu.sync_copy(cstage_v, cnt1_h.at[pl.ds(row0, rows_per_sub)])

    return sc_agg


def kernel(x, edge_index, W_in, b_in, W_l, b_l, W_r):
    n, d = x.shape
    h = W_in.shape[0]
    e = edge_index.shape[1]

    bn = 2000
    grid1 = n // bn
    feat = pl.pallas_call(
        _feat_body,
        grid=(grid1,),
        in_specs=[
            pl.BlockSpec((bn, d), lambda i: (i, 0)),
            pl.BlockSpec((h, d), lambda i: (0, 0)),
            pl.BlockSpec((1, h), lambda i: (0, 0)),
        ],
        out_specs=pl.BlockSpec((bn, h), lambda i: (i, 0)),
        out_shape=jax.ShapeDtypeStruct((n, h), jnp.float32),
    )(x, W_in, b_in.reshape(1, h))

    ktot = -(-e // (NS * BATCH))
    ktot = -(-ktot // 8) * 8
    k0 = ktot // 2
    k1 = ktot - k0
    e_pad = NS * ktot * BATCH
    n_pad = -(-(n + 1) // 256) * 256

    src = edge_index[0]
    dst = edge_index[1]
    src_p = jnp.concatenate([src, jnp.zeros((e_pad - e,), jnp.int32)])
    dst_p = jnp.concatenate([dst, jnp.full((e_pad - e,), n, jnp.int32)])
    comb = (src_p | (dst_p << 16)).reshape(NC, NS, k0, BATCH)

    sc_agg = _make_sc_agg(n_pad, k0, k1, h)
    agg_parts, cnt0, cnt1 = sc_agg(feat, comb, comb)

    cnt_nodes = (cnt0 + cnt1)[:n].reshape(n, 1)

    out_feat = pl.pallas_call(
        _out_body,
        grid=(grid1,),
        in_specs=[
            pl.BlockSpec((NC, bn, h), lambda i: (0, i, 0)),
            pl.BlockSpec((bn, 1), lambda i: (i, 0)),
            pl.BlockSpec((bn, h), lambda i: (i, 0)),
            pl.BlockSpec((h, h), lambda i: (0, 0)),
            pl.BlockSpec((1, h), lambda i: (0, 0)),
            pl.BlockSpec((h, h), lambda i: (0, 0)),
        ],
        out_specs=pl.BlockSpec((bn, h), lambda i: (i, 0)),
        out_shape=jax.ShapeDtypeStruct((n, h), jnp.float32),
    )(agg_parts, cnt_nodes, feat, W_l, b_l.reshape(1, h), W_r)

    return (feat, out_feat)


# speedup vs baseline: 1.0313x; 1.0313x over previous
import functools

import jax
import jax.numpy as jnp
from jax import lax
from jax.experimental import pallas as pl
from jax.experimental.pallas import tpu as pltpu
from jax.experimental.pallas import tpu_sc as plsc

NC = 2
NS = 16
BATCH = 128


def _feat_body(x_ref, w_ref, b_ref, o_ref):
    acc = lax.dot_general(x_ref[...], w_ref[...], (((1,), (1,)), ((), ())),
                          preferred_element_type=jnp.float32)
    o_ref[...] = jnp.maximum(acc + b_ref[...], 0.0)


def _out_body(a_ref, c_ref, f_ref, wl_ref, bl_ref, wr_ref, o_ref):
    agg = a_ref[0] + a_ref[1]
    mean = agg / jnp.maximum(c_ref[...], 1.0)
    o_ref[...] = (
        lax.dot_general(mean, wl_ref[...], (((1,), (1,)), ((), ())),
                        preferred_element_type=jnp.float32)
        + bl_ref[...]
        + lax.dot_general(f_ref[...], wr_ref[...], (((1,), (1,)), ((), ())),
                          preferred_element_type=jnp.float32)
    )


def _make_sc_agg(n_pad, k0, k1, d):
    rows_per_sub = n_pad // NS
    kmax = max(k0, k1)
    mesh = plsc.VectorSubcoreMesh(core_axis_name="c", subcore_axis_name="s")

    @functools.partial(
        pl.kernel,
        mesh=mesh,
        out_type=[
            jax.ShapeDtypeStruct((NC, n_pad, d), jnp.float32),
            jax.ShapeDtypeStruct((n_pad,), jnp.float32),
            jax.ShapeDtypeStruct((n_pad,), jnp.float32),
        ],
        scratch_types=[
            pltpu.VMEM((kmax, BATCH), jnp.int32),
            pltpu.VMEM((BATCH,), jnp.int32),
            pltpu.VMEM((BATCH,), jnp.int32),
            pltpu.VMEM((BATCH,), jnp.int32),
            pltpu.VMEM((BATCH,), jnp.int32),
            pltpu.VMEM((BATCH, d), jnp.float32),
            pltpu.VMEM((BATCH, d), jnp.float32),
            pltpu.VMEM((BATCH,), jnp.float32),
            pltpu.VMEM((rows_per_sub,), jnp.float32),
            pltpu.VMEM_SHARED((n_pad, d), jnp.float32),
            pltpu.VMEM_SHARED((n_pad,), jnp.float32),
            pltpu.SemaphoreType.DMA,
            pltpu.SemaphoreType.DMA,
        ],
    )
    def sc_agg(feat_h, comb0_h, comb1_h,
               agg_h, cnt0_h, cnt1_h,
               comb_v, srcA, dstA, srcB, dstB, rows_v, rows2_v, ones_v,
               cstage_v, acc_sh, cnt_sh, sem, sem2):
        c = lax.axis_index("c")
        s = lax.axis_index("s")
        row0 = s * rows_per_sub
        zv = jnp.zeros((16,), jnp.float32)
        ov = jnp.ones((16,), jnp.float32)

        def zfill(i, carry):
            for cc in range(d // 16):
                rows_v[i, pl.ds(cc * 16, 16)] = zv
            ones_v[pl.ds(i * 16, 16)] = ov
            return carry
        lax.fori_loop(0, BATCH // 16, zfill, 0)

        def zfill2(i, carry):
            for cc in range(d // 16):
                rows_v[i, pl.ds(cc * 16, 16)] = zv
            return carry
        lax.fori_loop(BATCH // 16, BATCH, zfill2, 0)

        full, rem = divmod(rows_per_sub, BATCH)
        for b in range(full):
            pltpu.sync_copy(rows_v, acc_sh.at[pl.ds(row0 + b * BATCH, BATCH)])
        if rem:
            pltpu.sync_copy(rows_v.at[pl.ds(0, rem)],
                            acc_sh.at[pl.ds(row0 + full * BATCH, rem)])
        for b in range(rows_per_sub // BATCH):
            pltpu.sync_copy(rows_v.at[0],
                            cnt_sh.at[pl.ds(row0 + b * BATCH, BATCH)])
        if rows_per_sub % BATCH:
            pltpu.sync_copy(rows_v.at[0, pl.ds(0, rows_per_sub % BATCH)],
                            cnt_sh.at[pl.ds(row0 + full * BATCH,
                                            rows_per_sub % BATCH)])

        plsc.subcore_barrier()

        def prep(j, srci, dsti):
            for b in range(BATCH // 16):
                cmb = comb_v[j, pl.ds(b * 16, 16)]
                srci[pl.ds(b * 16, 16)] = jnp.bitwise_and(cmb, 0xFFFF)
                dsti[pl.ds(b * 16, 16)] = lax.shift_right_logical(cmb, 16)

        def wait_gather(buf, s_):
            pltpu.make_async_copy(feat_h.at[pl.ds(0, BATCH)], buf, s_).wait()

        def scatter(buf, dsti):
            pltpu.sync_copy(buf, acc_sh.at[dsti], add=True)
            pltpu.sync_copy(ones_v, cnt_sh.at[dsti], add=True)

        pltpu.sync_copy(comb0_h.at[c, s], comb_v)
        prep(0, srcA, dstA)
        pltpu.async_copy(feat_h.at[srcA], rows_v, sem)

        def pair(jj, carry):
            j0 = 2 * jj
            prep(j0 + 1, srcB, dstB)
            pltpu.async_copy(feat_h.at[srcB], rows2_v, sem2)
            wait_gather(rows_v, sem)
            scatter(rows_v, dstA)
            prep(jnp.minimum(j0 + 2, kmax - 1), srcA, dstA)
            pltpu.async_copy(feat_h.at[srcA], rows_v, sem)
            wait_gather(rows2_v, sem2)
            scatter(rows2_v, dstB)
            return carry

        lax.fori_loop(0, kmax // 2, pair, 0)
        wait_gather(rows_v, sem)
        plsc.subcore_barrier()
        pltpu.sync_copy(acc_sh.at[pl.ds(row0, rows_per_sub)],
                        agg_h.at[c, pl.ds(row0, rows_per_sub)])

        pltpu.sync_copy(cnt_sh.at[pl.ds(row0, rows_per_sub)], cstage_v)

        @pl.when(c == 0)
        def _():
            pltpu.sync_copy(cstage_v, cnt0_h.at[pl.ds(row0, rows_per_sub)])

        @pl.when(c == 1)
        def _():
            pltpu.sync_copy(cstage_v, cnt1_h.at[pl.ds(row0, rows_per_sub)])

    return sc_agg


def kernel(x, edge_index, W_in, b_in, W_l, b_l, W_r):
    n, d = x.shape
    h = W_in.shape[0]
    e = edge_index.shape[1]

    bn = 2000
    grid1 = n // bn
    feat = pl.pallas_call(
        _feat_body,
        grid=(grid1,),
        in_specs=[
            pl.BlockSpec((bn, d), lambda i: (i, 0)),
            pl.BlockSpec((h, d), lambda i: (0, 0)),
            pl.BlockSpec((1, h), lambda i: (0, 0)),
        ],
        out_specs=pl.BlockSpec((bn, h), lambda i: (i, 0)),
        out_shape=jax.ShapeDtypeStruct((n, h), jnp.float32),
    )(x, W_in, b_in.reshape(1, h))

    ktot = -(-e // (NS * BATCH))
    ktot = -(-ktot // 8) * 8
    k0 = ktot // 2
    k1 = ktot - k0
    e_pad = NS * ktot * BATCH
    n_pad = -(-(n + 1) // 256) * 256

    src = edge_index[0]
    dst = edge_index[1]
    pad = e_pad - e
    src_p = jnp.concatenate([src, jnp.zeros((pad,), jnp.int32)])
    dst_p = jnp.concatenate(
        [dst, n + (jnp.arange(pad, dtype=jnp.int32) % (n_pad - n))])
    comb = (src_p | (dst_p << 16)).reshape(NC, NS, k0, BATCH)

    sc_agg = _make_sc_agg(n_pad, k0, k1, h)
    agg_parts, cnt0, cnt1 = sc_agg(feat, comb, comb)

    cnt_nodes = (cnt0 + cnt1)[:n].reshape(n, 1)

    out_feat = pl.pallas_call(
        _out_body,
        grid=(grid1,),
        in_specs=[
            pl.BlockSpec((NC, bn, h), lambda i: (0, i, 0)),
            pl.BlockSpec((bn, 1), lambda i: (i, 0)),
            pl.BlockSpec((bn, h), lambda i: (i, 0)),
            pl.BlockSpec((h, h), lambda i: (0, 0)),
            pl.BlockSpec((1, h), lambda i: (0, 0)),
            pl.BlockSpec((h, h), lambda i: (0, 0)),
        ],
        out_specs=pl.BlockSpec((bn, h), lambda i: (i, 0)),
        out_shape=jax.ShapeDtypeStruct((n, h), jnp.float32),
    )(agg_parts, cnt_nodes, feat, W_l, b_l.reshape(1, h), W_r)

    return (feat, out_feat)


# speedup vs baseline: 1.6377x; 1.5881x over previous
import functools

import jax
import jax.numpy as jnp
from jax import lax
from jax.experimental import pallas as pl
from jax.experimental.pallas import tpu as pltpu
from jax.experimental.pallas import tpu_sc as plsc

NC = 2
NS = 16
BATCH = 128


def _feat_body(x_ref, w_ref, b_ref, o_ref):
    acc = lax.dot_general(x_ref[...], w_ref[...], (((1,), (1,)), ((), ())),
                          preferred_element_type=jnp.float32)
    o_ref[...] = jnp.maximum(acc + b_ref[...], 0.0)


def _out_body(a_ref, c_ref, f_ref, wl_ref, bl_ref, wr_ref, o_ref):
    agg = a_ref[0] + a_ref[1]
    mean = agg / jnp.maximum(c_ref[...], 1.0)
    o_ref[...] = (
        lax.dot_general(mean, wl_ref[...], (((1,), (1,)), ((), ())),
                        preferred_element_type=jnp.float32)
        + bl_ref[...]
        + lax.dot_general(f_ref[...], wr_ref[...], (((1,), (1,)), ((), ())),
                          preferred_element_type=jnp.float32)
    )


def _make_sc_agg(n_pad, k_chunks, d):
    rows_per_sub = n_pad // NS
    kmax = k_chunks
    mesh = plsc.VectorSubcoreMesh(core_axis_name="c", subcore_axis_name="s")

    @functools.partial(
        pl.kernel,
        mesh=mesh,
        out_type=[
            jax.ShapeDtypeStruct((NC, n_pad, d), jnp.float32),
            jax.ShapeDtypeStruct((n_pad,), jnp.float32),
            jax.ShapeDtypeStruct((n_pad,), jnp.float32),
        ],
        scratch_types=[
            pltpu.VMEM((kmax, BATCH), jnp.int32),
            pltpu.VMEM((BATCH,), jnp.int32),
            pltpu.VMEM((BATCH,), jnp.int32),
            pltpu.VMEM((BATCH,), jnp.int32),
            pltpu.VMEM((BATCH,), jnp.int32),
            pltpu.VMEM((BATCH, d), jnp.float32),
            pltpu.VMEM((BATCH, d), jnp.float32),
            pltpu.VMEM((BATCH,), jnp.float32),
            pltpu.VMEM((rows_per_sub,), jnp.float32),
            pltpu.VMEM_SHARED((n_pad, d), jnp.float32),
            pltpu.VMEM_SHARED((n_pad,), jnp.float32),
            pltpu.SemaphoreType.DMA,
            pltpu.SemaphoreType.DMA,
        ],
    )
    def sc_agg(feat_h, comb0_h,
               agg_h, cnt0_h, cnt1_h,
               comb_v, srcA, dstA, srcB, dstB, rows_v, rows2_v, ones_v,
               cstage_v, acc_sh, cnt_sh, sem, sem2):
        c = lax.axis_index("c")
        s = lax.axis_index("s")
        row0 = s * rows_per_sub
        zv = jnp.zeros((16,), jnp.float32)
        ov = jnp.ones((16,), jnp.float32)

        def zfill(i, carry):
            for cc in range(d // 16):
                rows_v[i, pl.ds(cc * 16, 16)] = zv
            ones_v[pl.ds(i * 16, 16)] = ov
            return carry
        lax.fori_loop(0, BATCH // 16, zfill, 0)

        def zfill2(i, carry):
            for cc in range(d // 16):
                rows_v[i, pl.ds(cc * 16, 16)] = zv
            return carry
        lax.fori_loop(BATCH // 16, BATCH, zfill2, 0)

        full, rem = divmod(rows_per_sub, BATCH)
        for b in range(full):
            pltpu.sync_copy(rows_v, acc_sh.at[pl.ds(row0 + b * BATCH, BATCH)])
        if rem:
            pltpu.sync_copy(rows_v.at[pl.ds(0, rem)],
                            acc_sh.at[pl.ds(row0 + full * BATCH, rem)])
        for b in range(rows_per_sub // BATCH):
            pltpu.sync_copy(rows_v.at[0],
                            cnt_sh.at[pl.ds(row0 + b * BATCH, BATCH)])
        if rows_per_sub % BATCH:
            pltpu.sync_copy(rows_v.at[0, pl.ds(0, rows_per_sub % BATCH)],
                            cnt_sh.at[pl.ds(row0 + full * BATCH,
                                            rows_per_sub % BATCH)])

        plsc.subcore_barrier()

        def prep(j, srci, dsti):
            for b in range(BATCH // 16):
                cmb = comb_v[j, pl.ds(b * 16, 16)]
                srci[pl.ds(b * 16, 16)] = jnp.bitwise_and(cmb, 0xFFFF)
                dsti[pl.ds(b * 16, 16)] = lax.shift_right_logical(cmb, 16)

        def wait_gather(buf, s_):
            pltpu.make_async_copy(feat_h.at[pl.ds(0, BATCH)], buf, s_).wait()

        def scatter(buf, dsti):
            pltpu.sync_copy(buf, acc_sh.at[dsti], add=True)
            pltpu.sync_copy(ones_v, cnt_sh.at[dsti], add=True)

        pltpu.sync_copy(comb0_h.at[c, s], comb_v)
        prep(0, srcA, dstA)
        pltpu.async_copy(feat_h.at[srcA], rows_v, sem)

        def pair(jj, carry):
            j0 = 2 * jj
            prep(j0 + 1, srcB, dstB)
            pltpu.async_copy(feat_h.at[srcB], rows2_v, sem2)
            wait_gather(rows_v, sem)
            scatter(rows_v, dstA)

            @pl.when(j0 + 2 < kmax)
            def _():
                prep(j0 + 2, srcA, dstA)
                pltpu.async_copy(feat_h.at[srcA], rows_v, sem)
            wait_gather(rows2_v, sem2)
            scatter(rows2_v, dstB)
            return carry

        lax.fori_loop(0, kmax // 2, pair, 0)
        if kmax % 2:
            wait_gather(rows_v, sem)
            scatter(rows_v, dstA)
        plsc.subcore_barrier()
        pltpu.sync_copy(acc_sh.at[pl.ds(row0, rows_per_sub)],
                        agg_h.at[c, pl.ds(row0, rows_per_sub)])

        pltpu.sync_copy(cnt_sh.at[pl.ds(row0, rows_per_sub)], cstage_v)

        @pl.when(c == 0)
        def _():
            pltpu.sync_copy(cstage_v, cnt0_h.at[pl.ds(row0, rows_per_sub)])

        @pl.when(c == 1)
        def _():
            pltpu.sync_copy(cstage_v, cnt1_h.at[pl.ds(row0, rows_per_sub)])

    return sc_agg


def kernel(x, edge_index, W_in, b_in, W_l, b_l, W_r):
    n, d = x.shape
    h = W_in.shape[0]
    e = edge_index.shape[1]

    bn = 2000
    grid1 = n // bn
    feat = pl.pallas_call(
        _feat_body,
        grid=(grid1,),
        in_specs=[
            pl.BlockSpec((bn, d), lambda i: (i, 0)),
            pl.BlockSpec((h, d), lambda i: (0, 0)),
            pl.BlockSpec((1, h), lambda i: (0, 0)),
        ],
        out_specs=pl.BlockSpec((bn, h), lambda i: (i, 0)),
        out_shape=jax.ShapeDtypeStruct((n, h), jnp.float32),
    )(x, W_in, b_in.reshape(1, h))

    nw = NC * NS
    k_chunks = -(-e // (nw * BATCH))
    e_pad = nw * k_chunks * BATCH
    n_pad = -(-(n + 1) // 256) * 256

    src = edge_index[0]
    dst = edge_index[1]
    pad = e_pad - e
    src_p = jnp.concatenate([src, jnp.zeros((pad,), jnp.int32)])
    dst_p = jnp.concatenate(
        [dst, n + (jnp.arange(pad, dtype=jnp.int32) % (n_pad - n))])
    comb = (src_p | (dst_p << 16)).reshape(NC, NS, k_chunks, BATCH)

    sc_agg = _make_sc_agg(n_pad, k_chunks, h)
    agg_parts, cnt0, cnt1 = sc_agg(feat, comb)

    cnt_nodes = (cnt0 + cnt1)[:n].reshape(n, 1)

    out_feat = pl.pallas_call(
        _out_body,
        grid=(grid1,),
        in_specs=[
            pl.BlockSpec((NC, bn, h), lambda i: (0, i, 0)),
            pl.BlockSpec((bn, 1), lambda i: (i, 0)),
            pl.BlockSpec((bn, h), lambda i: (i, 0)),
            pl.BlockSpec((h, h), lambda i: (0, 0)),
            pl.BlockSpec((1, h), lambda i: (0, 0)),
            pl.BlockSpec((h, h), lambda i: (0, 0)),
        ],
        out_specs=pl.BlockSpec((bn, h), lambda i: (i, 0)),
        out_shape=jax.ShapeDtypeStruct((n, h), jnp.float32),
    )(agg_parts, cnt_nodes, feat, W_l, b_l.reshape(1, h), W_r)

    return (feat, out_feat)
